# baseline (speedup 1.0000x reference)
import functools

import jax
import jax.numpy as jnp
from jax import lax
from jax.experimental import pallas as pl
from jax.experimental.pallas import tpu as pltpu
from jax.experimental.pallas import tpu_sc as plsc

_N = 10000
_E = 160000
_H = 256
_G = 64
_T = 4

_NTILES = 16
_NCORES = 2
_K = 64
_NB = 160
_C = 128
_EPT = _K * _NB
_EPAD = _EPT * _NTILES
_NACC = 10240

_R = 1000
_NBLK = _N // _R


def _sc_aggregate(hs, srcs, dsts, zeros, out, src_v, dst_v, rows_v, acc,
                  gs0, gs1, ss0, ss1):
    c = lax.axis_index("c")
    s = lax.axis_index("s")
    pltpu.sync_copy(zeros, rows_v)
    for j in range(_NACC // _C // _NTILES):
        pltpu.sync_copy(rows_v, acc.at[pl.ds((s + _NTILES * j) * _C, _C)])
    plsc.subcore_barrier()
    table = hs.at[c]
    half = _NB // 2
    npair = half // 2
    buf0 = rows_v.at[pl.ds(0, _K)]
    buf1 = rows_v.at[pl.ds(_K, _K)]

    def gather(j, buf, sem):
        return pltpu.async_copy(table.at[src_v.at[j]], buf, sem)

    def gather_wait(j, buf, sem):
        pltpu.make_async_copy(table.at[src_v.at[j]], buf, sem).wait()

    def scat(j, buf, sem):
        return pltpu.async_copy(buf, acc.at[dst_v.at[j]], sem, add=True)

    def scat_wait(j, buf, sem):
        pltpu.make_async_copy(buf, acc.at[dst_v.at[j]], sem).wait()

    def pair_body(j2, carry):
        j = 2 * j2
        gather_wait(j, buf0, gs0)
        scat(j, buf0, ss0)

        @pl.when(j2 > 0)
        def _():
            scat_wait(j - 1, buf1, ss1)

        gather(j + 1, buf1, gs1)
        gather_wait(j + 1, buf1, gs1)
        scat(j + 1, buf1, ss1)
        scat_wait(j, buf0, ss0)

        @pl.when(j2 + 1 < npair)
        def _():
            gather(j + 2, buf0, gs0)

        return carry

    for h in range(2):
        pltpu.sync_copy(srcs.at[s].at[pl.ds(h * half, half)], src_v)
        pltpu.sync_copy(dsts.at[s].at[pl.ds(h * half, half)], dst_v)
        gather(0, buf0, gs0)
        lax.fori_loop(0, npair, pair_body, 0)
        scat_wait(half - 1, buf1, ss1)
    plsc.subcore_barrier()
    nfull = _N // _C
    for j in range(5):
        idx = s + _NTILES * j

        @pl.when(idx < nfull)
        def _():
            base = pl.multiple_of(idx * _C, _C)
            pltpu.sync_copy(acc.at[pl.ds(base, _C)], rows_v)
            pltpu.sync_copy(rows_v, out.at[c].at[pl.ds(base, _C)])

    tail = _N - nfull * _C

    @pl.when(s == _NTILES - 1)
    def _():
        pltpu.sync_copy(acc.at[pl.ds(nfull * _C, tail)],
                        rows_v.at[pl.ds(0, tail)])
        pltpu.sync_copy(rows_v.at[pl.ds(0, tail)],
                        out.at[c].at[pl.ds(nfull * _C, tail)])


def _make_sc_agg():
    mesh = plsc.VectorSubcoreMesh(core_axis_name="c", subcore_axis_name="s")
    return pl.kernel(
        _sc_aggregate,
        out_type=jax.ShapeDtypeStruct((_NCORES, _N, 128), jnp.float32),
        mesh=mesh,
        scratch_types=[
            pltpu.VMEM((_NB // 2, _K), jnp.int32),
            pltpu.VMEM((_NB // 2, _K), jnp.int32),
            pltpu.VMEM((2 * _K, 128), jnp.float32),
            pltpu.VMEM_SHARED((_NACC, 128), jnp.float32),
            pltpu.SemaphoreType.DMA,
            pltpu.SemaphoreType.DMA,
            pltpu.SemaphoreType.DMA,
            pltpu.SemaphoreType.DMA,
        ],
    )


def _bn_scale_shift(s1_ref, s2_ref, g_ref, be_ref):
    ninv = 1.0 / _N
    mean = s1_ref[...] * ninv
    var = s2_ref[...] * ninv - mean * mean
    scale = g_ref[...] * lax.rsqrt(var + 1e-5)
    shift = be_ref[...] - mean * scale
    return scale, shift


def _phase0(xs_ref, agg_ref, w1_ref, b1_ref, yv_ref, s1_ref, s2_ref):
    i = pl.program_id(1)
    xa = jnp.concatenate(
        [xs_ref[0] + agg_ref[0], xs_ref[1] + agg_ref[1]], axis=1)
    y = jnp.dot(xa, w1_ref[...], preferred_element_type=jnp.float32) + b1_ref[...]
    yv_ref[pl.ds(i * _R, _R), :] = y

    @pl.when(i == 0)
    def _():
        s1_ref[...] = jnp.zeros_like(s1_ref)
        s2_ref[...] = jnp.zeros_like(s2_ref)

    s1_ref[...] += jnp.sum(y, axis=0, keepdims=True)
    s2_ref[...] += jnp.sum(y * y, axis=0, keepdims=True)


def _mlp_body(xs_ref, agg_ref, w1_ref, b1_ref, g_ref, be_ref, w2_ref, b2_ref,
              hs_ref, yv_ref, s1_ref, s2_ref):
    p = pl.program_id(0)
    i = pl.program_id(1)

    @pl.when(p == 0)
    def _():
        _phase0(xs_ref, agg_ref, w1_ref, b1_ref, yv_ref, s1_ref, s2_ref)

    @pl.when(p == 1)
    def _():
        scale, shift = _bn_scale_shift(s1_ref, s2_ref, g_ref, be_ref)
        h = jnp.maximum(yv_ref[pl.ds(i * _R, _R), :] * scale + shift, 0.0)
        o = jnp.dot(h, w2_ref[...], preferred_element_type=jnp.float32) + b2_ref[...]
        o = jnp.maximum(o, 0.0)
        hs_ref[0] = o[:, :128]
        hs_ref[1] = o[:, 128:]


def _mlp(xs, agg, w1, b1r, gr, ber, w2, b2r):
    def in_map(p, i):
        return (0, i * (1 - p), 0)

    def out_map(p, i):
        return (0, i * p, 0)

    return pl.pallas_call(
        _mlp_body,
        grid=(2, _NBLK),
        in_specs=[
            pl.BlockSpec((_NCORES, _R, 128), in_map),
            pl.BlockSpec((_NCORES, _R, 128), in_map),
            pl.BlockSpec((_H, _H), lambda p, i: (0, 0)),
            pl.BlockSpec((1, _H), lambda p, i: (0, 0)),
            pl.BlockSpec((1, _H), lambda p, i: (0, 0)),
            pl.BlockSpec((1, _H), lambda p, i: (0, 0)),
            pl.BlockSpec((_H, _H), lambda p, i: (0, 0)),
            pl.BlockSpec((1, _H), lambda p, i: (0, 0)),
        ],
        out_specs=pl.BlockSpec((_NCORES, _R, 128), out_map),
        out_shape=jax.ShapeDtypeStruct((_NCORES, _N, 128), jnp.float32),
        scratch_shapes=[
            pltpu.VMEM((_N, _H), jnp.float32),
            pltpu.VMEM((1, _H), jnp.float32),
            pltpu.VMEM((1, _H), jnp.float32),
        ],
    )(xs, agg, w1, b1r, gr, ber, w2, b2r)


def _mlp3_body(xs_ref, agg_ref, w1_ref, b1_ref, g_ref, be_ref, w2_ref, b2_ref,
               batch_ref, rt_ref, whm_ref, bh_ref, out_ref,
               yv_ref, s1_ref, s2_ref, acc_ref):
    p = pl.program_id(0)
    i = pl.program_id(1)

    @pl.when(p == 0)
    def _():
        _phase0(xs_ref, agg_ref, w1_ref, b1_ref, yv_ref, s1_ref, s2_ref)

    @pl.when(p == 1)
    def _():
        scale, shift = _bn_scale_shift(s1_ref, s2_ref, g_ref, be_ref)
        h = jnp.maximum(yv_ref[pl.ds(i * _R, _R), :] * scale + shift, 0.0)
        o = jnp.dot(h, w2_ref[...], preferred_element_type=jnp.float32) + b2_ref[...]
        o = jnp.maximum(o, 0.0)

        @pl.when(i == 0)
        def _():
            acc_ref[...] = jnp.zeros_like(acc_ref)

        onehot = (batch_ref[...] ==
                  lax.broadcasted_iota(jnp.int32, (_R, _G), 1)
                  ).astype(jnp.float32)
        acc_ref[...] += lax.dot_general(
            onehot, o, (((0,), (0,)), ((), ())),
            preferred_element_type=jnp.float32)

        @pl.when(i == _NBLK - 1)
        def _():
            pooled = acc_ref[...]
            proj = jnp.dot(pooled, whm_ref[...],
                           preferred_element_type=jnp.float32)
            sel = (rt_ref[...] ==
                   lax.broadcasted_iota(jnp.int32, (_G, _T), 1)
                   ).astype(jnp.float32)
            res = jnp.sum((proj + bh_ref[...]) * sel, axis=1)
            out_ref[...] = res[None, :]


def _mlp3(xs, agg, w1, b1r, gr, ber, w2, b2r, batch2, rt2, whm, bhr):
    def in_map(p, i):
        return (0, i * (1 - p), 0)

    return pl.pallas_call(
        _mlp3_body,
        grid=(2, _NBLK),
        in_specs=[
            pl.BlockSpec((_NCORES, _R, 128), in_map),
            pl.BlockSpec((_NCORES, _R, 128), in_map),
            pl.BlockSpec((_H, _H), lambda p, i: (0, 0)),
            pl.BlockSpec((1, _H), lambda p, i: (0, 0)),
            pl.BlockSpec((1, _H), lambda p, i: (0, 0)),
            pl.BlockSpec((1, _H), lambda p, i: (0, 0)),
            pl.BlockSpec((_H, _H), lambda p, i: (0, 0)),
            pl.BlockSpec((1, _H), lambda p, i: (0, 0)),
            pl.BlockSpec((_R, 1), lambda p, i: (i * p, 0)),
            pl.BlockSpec((_G, 1), lambda p, i: (0, 0)),
            pl.BlockSpec((_H, _T), lambda p, i: (0, 0)),
            pl.BlockSpec((1, _T), lambda p, i: (0, 0)),
        ],
        out_specs=pl.BlockSpec((1, _G), lambda p, i: (0, 0)),
        out_shape=jax.ShapeDtypeStruct((1, _G), jnp.float32),
        scratch_shapes=[
            pltpu.VMEM((_N, _H), jnp.float32),
            pltpu.VMEM((1, _H), jnp.float32),
            pltpu.VMEM((1, _H), jnp.float32),
            pltpu.VMEM((_G, _H), jnp.float32),
        ],
    )(xs, agg, w1, b1r, gr, ber, w2, b2r, batch2, rt2, whm, bhr)


def kernel(x, edge_index, batch, r_target, W1_1, b1_1, g_1, be_1, W2_1, b2_1,
           W1_2, b1_2, g_2, be_2, W2_2, b2_2, W1_3, b1_3, g_3, be_3, W2_3,
           b2_3, Wh, bh):
    src = edge_index[0]
    dst = edge_index[1]
    pad = _EPAD - _E
    srcs = jnp.concatenate([src, jnp.zeros((pad,), jnp.int32)]).reshape(
        _NTILES, _NB, _K)
    dsts = jnp.concatenate([dst, jnp.full((pad,), _N, jnp.int32)]).reshape(
        _NTILES, _NB, _K)
    zeros = jnp.zeros((_C, 128), jnp.float32)

    hs = jnp.stack([x[:, :128], x[:, 128:]])
    sc_agg = _make_sc_agg()

    for (w1, b1, g, be, w2, b2) in [
        (W1_1, b1_1, g_1, be_1, W2_1, b2_1),
        (W1_2, b1_2, g_2, be_2, W2_2, b2_2),
    ]:
        agg = sc_agg(hs, srcs, dsts, zeros)
        hs = _mlp(hs, agg, w1, b1[None], g[None], be[None], w2, b2[None])

    agg = sc_agg(hs, srcs, dsts, zeros)
    whm = Wh[:, :, 0].T
    bhr = bh[:, 0][None]
    out = _mlp3(hs, agg, W1_3, b1_3[None], g_3[None], be_3[None], W2_3,
                b2_3[None], batch[:, None], r_target[:, None], whm, bhr)
    return out.reshape(_G)


# speedup vs baseline: 1.0533x; 1.0533x over previous
import functools

import jax
import jax.numpy as jnp
from jax import lax
from jax.experimental import pallas as pl
from jax.experimental.pallas import tpu as pltpu
from jax.experimental.pallas import tpu_sc as plsc

_N = 10000
_E = 160000
_H = 256
_G = 64
_T = 4

_NTILES = 16
_NCORES = 2
_K = 128
_NB = 80
_C = 128
_EPT = _K * _NB
_EPAD = _EPT * _NTILES
_NACC = 10240

_R = 1000
_NBLK = _N // _R


def _sc_aggregate(hs, srcs, dsts, zeros, out, src_v, dst_v, rows_v, acc,
                  gs0, gs1, ss0, ss1):
    c = lax.axis_index("c")
    s = lax.axis_index("s")
    zbuf = rows_v.at[pl.ds(0, _C)]
    pltpu.sync_copy(zeros, zbuf)
    for j in range(_NACC // _C // _NTILES):
        pltpu.sync_copy(zbuf, acc.at[pl.ds((s + _NTILES * j) * _C, _C)])
    plsc.subcore_barrier()
    table = hs.at[c]
    half = _NB // 2
    npair = half // 2
    buf0 = rows_v.at[pl.ds(0, _K)]
    buf1 = rows_v.at[pl.ds(_K, _K)]

    def gather(j, buf, sem):
        return pltpu.async_copy(table.at[src_v.at[j]], buf, sem)

    def gather_wait(j, buf, sem):
        pltpu.make_async_copy(table.at[src_v.at[j]], buf, sem).wait()

    def scat(j, buf, sem):
        return pltpu.async_copy(buf, acc.at[dst_v.at[j]], sem, add=True)

    def scat_wait(j, buf, sem):
        pltpu.make_async_copy(buf, acc.at[dst_v.at[j]], sem).wait()

    def pair_body(j2, carry):
        j = 2 * j2
        gather_wait(j, buf0, gs0)
        scat(j, buf0, ss0)

        @pl.when(j2 > 0)
        def _():
            scat_wait(j - 1, buf1, ss1)

        gather(j + 1, buf1, gs1)
        gather_wait(j + 1, buf1, gs1)
        scat(j + 1, buf1, ss1)
        scat_wait(j, buf0, ss0)

        @pl.when(j2 + 1 < npair)
        def _():
            gather(j + 2, buf0, gs0)

        return carry

    for h in range(2):
        pltpu.sync_copy(srcs.at[s].at[pl.ds(h * half, half)], src_v)
        pltpu.sync_copy(dsts.at[s].at[pl.ds(h * half, half)], dst_v)
        gather(0, buf0, gs0)
        lax.fori_loop(0, npair, pair_body, 0)
        scat_wait(half - 1, buf1, ss1)
    plsc.subcore_barrier()
    nfull = _N // _C
    for j in range(5):
        idx = s + _NTILES * j

        @pl.when(idx < nfull)
        def _():
            base = pl.multiple_of(idx * _C, _C)
            pltpu.sync_copy(acc.at[pl.ds(base, _C)], zbuf)
            pltpu.sync_copy(zbuf, out.at[c].at[pl.ds(base, _C)])

    tail = _N - nfull * _C

    @pl.when(s == _NTILES - 1)
    def _():
        pltpu.sync_copy(acc.at[pl.ds(nfull * _C, tail)],
                        zbuf.at[pl.ds(0, tail)])
        pltpu.sync_copy(zbuf.at[pl.ds(0, tail)],
                        out.at[c].at[pl.ds(nfull * _C, tail)])


def _make_sc_agg():
    mesh = plsc.VectorSubcoreMesh(core_axis_name="c", subcore_axis_name="s")
    return pl.kernel(
        _sc_aggregate,
        out_type=jax.ShapeDtypeStruct((_NCORES, _N, 128), jnp.float32),
        mesh=mesh,
        scratch_types=[
            pltpu.VMEM((_NB // 2, _K), jnp.int32),
            pltpu.VMEM((_NB // 2, _K), jnp.int32),
            pltpu.VMEM((2 * _K, 128), jnp.float32),
            pltpu.VMEM_SHARED((_NACC, 128), jnp.float32),
            pltpu.SemaphoreType.DMA,
            pltpu.SemaphoreType.DMA,
            pltpu.SemaphoreType.DMA,
            pltpu.SemaphoreType.DMA,
        ],
    )


def _bn_scale_shift(s1_ref, s2_ref, g_ref, be_ref):
    ninv = 1.0 / _N
    mean = s1_ref[...] * ninv
    var = s2_ref[...] * ninv - mean * mean
    scale = g_ref[...] * lax.rsqrt(var + 1e-5)
    shift = be_ref[...] - mean * scale
    return scale, shift


def _phase0(xs_ref, agg_ref, w1_ref, b1_ref, yv_ref, s1_ref, s2_ref):
    i = pl.program_id(1)
    xa = jnp.concatenate(
        [xs_ref[0] + agg_ref[0], xs_ref[1] + agg_ref[1]], axis=1)
    y = jnp.dot(xa, w1_ref[...], preferred_element_type=jnp.float32) + b1_ref[...]
    yv_ref[pl.ds(i * _R, _R), :] = y

    @pl.when(i == 0)
    def _():
        s1_ref[...] = jnp.zeros_like(s1_ref)
        s2_ref[...] = jnp.zeros_like(s2_ref)

    s1_ref[...] += jnp.sum(y, axis=0, keepdims=True)
    s2_ref[...] += jnp.sum(y * y, axis=0, keepdims=True)


def _mlp_body(xs_ref, agg_ref, w1_ref, b1_ref, g_ref, be_ref, w2_ref, b2_ref,
              hs_ref, yv_ref, s1_ref, s2_ref):
    p = pl.program_id(0)
    i = pl.program_id(1)

    @pl.when(p == 0)
    def _():
        _phase0(xs_ref, agg_ref, w1_ref, b1_ref, yv_ref, s1_ref, s2_ref)

    @pl.when(p == 1)
    def _():
        scale, shift = _bn_scale_shift(s1_ref, s2_ref, g_ref, be_ref)
        h = jnp.maximum(yv_ref[pl.ds(i * _R, _R), :] * scale + shift, 0.0)
        o = jnp.dot(h, w2_ref[...], preferred_element_type=jnp.float32) + b2_ref[...]
        o = jnp.maximum(o, 0.0)
        hs_ref[0] = o[:, :128]
        hs_ref[1] = o[:, 128:]


def _mlp(xs, agg, w1, b1r, gr, ber, w2, b2r):
    def in_map(p, i):
        return (0, i * (1 - p), 0)

    def out_map(p, i):
        return (0, i * p, 0)

    return pl.pallas_call(
        _mlp_body,
        grid=(2, _NBLK),
        in_specs=[
            pl.BlockSpec((_NCORES, _R, 128), in_map),
            pl.BlockSpec((_NCORES, _R, 128), in_map),
            pl.BlockSpec((_H, _H), lambda p, i: (0, 0)),
            pl.BlockSpec((1, _H), lambda p, i: (0, 0)),
            pl.BlockSpec((1, _H), lambda p, i: (0, 0)),
            pl.BlockSpec((1, _H), lambda p, i: (0, 0)),
            pl.BlockSpec((_H, _H), lambda p, i: (0, 0)),
            pl.BlockSpec((1, _H), lambda p, i: (0, 0)),
        ],
        out_specs=pl.BlockSpec((_NCORES, _R, 128), out_map),
        out_shape=jax.ShapeDtypeStruct((_NCORES, _N, 128), jnp.float32),
        scratch_shapes=[
            pltpu.VMEM((_N, _H), jnp.float32),
            pltpu.VMEM((1, _H), jnp.float32),
            pltpu.VMEM((1, _H), jnp.float32),
        ],
    )(xs, agg, w1, b1r, gr, ber, w2, b2r)


def _mlp3_body(xs_ref, agg_ref, w1_ref, b1_ref, g_ref, be_ref, w2_ref, b2_ref,
               batch_ref, rt_ref, whm_ref, bh_ref, out_ref,
               yv_ref, s1_ref, s2_ref, acc_ref):
    p = pl.program_id(0)
    i = pl.program_id(1)

    @pl.when(p == 0)
    def _():
        _phase0(xs_ref, agg_ref, w1_ref, b1_ref, yv_ref, s1_ref, s2_ref)

    @pl.when(p == 1)
    def _():
        scale, shift = _bn_scale_shift(s1_ref, s2_ref, g_ref, be_ref)
        h = jnp.maximum(yv_ref[pl.ds(i * _R, _R), :] * scale + shift, 0.0)
        o = jnp.dot(h, w2_ref[...], preferred_element_type=jnp.float32) + b2_ref[...]
        o = jnp.maximum(o, 0.0)

        @pl.when(i == 0)
        def _():
            acc_ref[...] = jnp.zeros_like(acc_ref)

        onehot = (batch_ref[...] ==
                  lax.broadcasted_iota(jnp.int32, (_R, _G), 1)
                  ).astype(jnp.float32)
        acc_ref[...] += lax.dot_general(
            onehot, o, (((0,), (0,)), ((), ())),
            preferred_element_type=jnp.float32)

        @pl.when(i == _NBLK - 1)
        def _():
            pooled = acc_ref[...]
            proj = jnp.dot(pooled, whm_ref[...],
                           preferred_element_type=jnp.float32)
            sel = (rt_ref[...] ==
                   lax.broadcasted_iota(jnp.int32, (_G, _T), 1)
                   ).astype(jnp.float32)
            res = jnp.sum((proj + bh_ref[...]) * sel, axis=1)
            out_ref[...] = res[None, :]


def _mlp3(xs, agg, w1, b1r, gr, ber, w2, b2r, batch2, rt2, whm, bhr):
    def in_map(p, i):
        return (0, i * (1 - p), 0)

    return pl.pallas_call(
        _mlp3_body,
        grid=(2, _NBLK),
        in_specs=[
            pl.BlockSpec((_NCORES, _R, 128), in_map),
            pl.BlockSpec((_NCORES, _R, 128), in_map),
            pl.BlockSpec((_H, _H), lambda p, i: (0, 0)),
            pl.BlockSpec((1, _H), lambda p, i: (0, 0)),
            pl.BlockSpec((1, _H), lambda p, i: (0, 0)),
            pl.BlockSpec((1, _H), lambda p, i: (0, 0)),
            pl.BlockSpec((_H, _H), lambda p, i: (0, 0)),
            pl.BlockSpec((1, _H), lambda p, i: (0, 0)),
            pl.BlockSpec((_R, 1), lambda p, i: (i * p, 0)),
            pl.BlockSpec((_G, 1), lambda p, i: (0, 0)),
            pl.BlockSpec((_H, _T), lambda p, i: (0, 0)),
            pl.BlockSpec((1, _T), lambda p, i: (0, 0)),
        ],
        out_specs=pl.BlockSpec((1, _G), lambda p, i: (0, 0)),
        out_shape=jax.ShapeDtypeStruct((1, _G), jnp.float32),
        scratch_shapes=[
            pltpu.VMEM((_N, _H), jnp.float32),
            pltpu.VMEM((1, _H), jnp.float32),
            pltpu.VMEM((1, _H), jnp.float32),
            pltpu.VMEM((_G, _H), jnp.float32),
        ],
    )(xs, agg, w1, b1r, gr, ber, w2, b2r, batch2, rt2, whm, bhr)


def kernel(x, edge_index, batch, r_target, W1_1, b1_1, g_1, be_1, W2_1, b2_1,
           W1_2, b1_2, g_2, be_2, W2_2, b2_2, W1_3, b1_3, g_3, be_3, W2_3,
           b2_3, Wh, bh):
    src = edge_index[0]
    dst = edge_index[1]
    pad = _EPAD - _E
    srcs = jnp.concatenate([src, jnp.zeros((pad,), jnp.int32)]).reshape(
        _NTILES, _NB, _K)
    dsts = jnp.concatenate([dst, jnp.full((pad,), _N, jnp.int32)]).reshape(
        _NTILES, _NB, _K)
    zeros = jnp.zeros((_C, 128), jnp.float32)

    hs = jnp.stack([x[:, :128], x[:, 128:]])
    sc_agg = _make_sc_agg()

    for (w1, b1, g, be, w2, b2) in [
        (W1_1, b1_1, g_1, be_1, W2_1, b2_1),
        (W1_2, b1_2, g_2, be_2, W2_2, b2_2),
    ]:
        agg = sc_agg(hs, srcs, dsts, zeros)
        hs = _mlp(hs, agg, w1, b1[None], g[None], be[None], w2, b2[None])

    agg = sc_agg(hs, srcs, dsts, zeros)
    whm = Wh[:, :, 0].T
    bhr = bh[:, 0][None]
    out = _mlp3(hs, agg, W1_3, b1_3[None], g_3[None], be_3[None], W2_3,
                b2_3[None], batch[:, None], r_target[:, None], whm, bhr)
    return out.reshape(_G)
